# baseline (speedup 1.0000x reference)
import functools

import jax
import jax.numpy as jnp
from jax.experimental import pallas as pl
from jax.experimental.pallas import tpu as pltpu


def _round_up(x, m):
    return ((x + m - 1) // m) * m


def _pipe_kernel(x_any, w_ref, b_ref, o_any,
                 w_bf, x_buf, o_buf, in_sem, out_sem,
                 *, block, n_steps):
    core = pl.program_id(0)
    base = core * (n_steps * block)

    def dma_in(slot, step):
        pltpu.make_async_copy(
            x_any.at[pl.ds(base + step * block, block), :],
            x_buf.at[slot], in_sem.at[slot]).start()

    def wait_in(slot):
        pltpu.make_async_copy(
            x_any.at[pl.ds(base, block), :],
            x_buf.at[slot], in_sem.at[slot]).wait()

    def dma_out(slot, step):
        pltpu.make_async_copy(
            o_buf.at[slot],
            o_any.at[pl.ds(base + step * block, block), :],
            out_sem.at[slot]).start()

    def wait_out(slot):
        pltpu.make_async_copy(
            o_buf.at[slot],
            o_any.at[pl.ds(base, block), :],
            out_sem.at[slot]).wait()

    dma_in(0, 0)
    w_bf[...] = w_ref[...].astype(jnp.bfloat16)

    def body(step, _):
        cur = jax.lax.rem(step, 2)
        nxt = jax.lax.rem(step + 1, 2)

        @pl.when(step + 1 < n_steps)
        def _():
            dma_in(nxt, step + 1)

        wait_in(cur)

        @pl.when(step >= 2)
        def _():
            wait_out(cur)

        xb = x_buf[cur].astype(jnp.bfloat16)
        o_buf[cur] = (
            jnp.dot(xb, w_bf[...], preferred_element_type=jnp.float32)
            + b_ref[...]
        )
        dma_out(cur, step)
        return ()

    jax.lax.fori_loop(0, n_steps, body, ())
    wait_out(jax.lax.rem(n_steps - 2, 2))
    wait_out(jax.lax.rem(n_steps - 1, 2))


@jax.jit
def _affine(x, w, bias):
    batch, in_dim = x.shape
    _, n = w.shape

    block = 512
    n_cores = 2
    m_pad = _round_up(batch, block * n_cores)
    x_p = x if m_pad == batch else jnp.pad(x, ((0, m_pad - batch), (0, 0)))
    n_steps = m_pad // (block * n_cores)

    out = pl.pallas_call(
        functools.partial(_pipe_kernel, block=block, n_steps=n_steps),
        out_shape=jax.ShapeDtypeStruct((m_pad, n), jnp.float32),
        grid=(n_cores,),
        in_specs=[
            pl.BlockSpec(memory_space=pl.ANY),
            pl.BlockSpec((in_dim, n), lambda i: (0, 0)),
            pl.BlockSpec((1, n), lambda i: (0, 0)),
        ],
        out_specs=pl.BlockSpec(memory_space=pl.ANY),
        scratch_shapes=[
            pltpu.VMEM((in_dim, n), jnp.bfloat16),
            pltpu.VMEM((2, block, in_dim), jnp.float32),
            pltpu.VMEM((2, block, n), jnp.float32),
            pltpu.SemaphoreType.DMA((2,)),
            pltpu.SemaphoreType.DMA((2,)),
        ],
        compiler_params=pltpu.CompilerParams(
            dimension_semantics=("parallel",),
            vmem_limit_bytes=56 * 1024 * 1024,
        ),
    )(x_p, w, bias)

    return out[:batch] if m_pad != batch else out


def kernel(x, wt_padded, bias_padded):
    return _affine(x, wt_padded, bias_padded)


# speedup vs baseline: 1.0821x; 1.0821x over previous
import functools

import jax
import jax.numpy as jnp
from jax.experimental import pallas as pl
from jax.experimental.pallas import tpu as pltpu


def _round_up(x, m):
    return ((x + m - 1) // m) * m


def _pipe_kernel(x_any, w_ref, b_ref, o_any,
                 w_bf, x_buf, o_buf, in_sem, out_sem,
                 *, block, n_steps):
    core = pl.program_id(0)
    base = core * (n_steps * block)

    def dma_in(slot, step):
        pltpu.make_async_copy(
            x_any.at[pl.ds(base + step * block, block), :],
            x_buf.at[slot], in_sem.at[slot]).start()

    def wait_in(slot):
        pltpu.make_async_copy(
            x_any.at[pl.ds(base, block), :],
            x_buf.at[slot], in_sem.at[slot]).wait()

    def dma_out(slot, step):
        pltpu.make_async_copy(
            o_buf.at[slot],
            o_any.at[pl.ds(base + step * block, block), :],
            out_sem.at[slot]).start()

    def wait_out(slot):
        pltpu.make_async_copy(
            o_buf.at[slot],
            o_any.at[pl.ds(base, block), :],
            out_sem.at[slot]).wait()

    dma_in(0, 0)
    w_bf[...] = w_ref[...].astype(jnp.bfloat16)

    def body(step, _):
        cur = jax.lax.rem(step, 2)
        nxt = jax.lax.rem(step + 1, 2)

        @pl.when(step + 1 < n_steps)
        def _():
            dma_in(nxt, step + 1)

        wait_in(cur)

        @pl.when(step >= 2)
        def _():
            wait_out(cur)

        xb = x_buf[cur].astype(jnp.bfloat16)
        o_buf[cur] = (
            jnp.dot(xb, w_bf[...], preferred_element_type=jnp.float32)
            + b_ref[...]
        )
        dma_out(cur, step)
        return ()

    jax.lax.fori_loop(0, n_steps, body, ())
    wait_out(jax.lax.rem(n_steps - 2, 2))
    wait_out(jax.lax.rem(n_steps - 1, 2))


@jax.jit
def _affine(x, w, bias):
    batch, in_dim = x.shape
    _, n = w.shape

    block = 1024
    n_cores = 2
    m_pad = _round_up(batch, block * n_cores)
    x_p = x if m_pad == batch else jnp.pad(x, ((0, m_pad - batch), (0, 0)))
    n_steps = m_pad // (block * n_cores)

    out = pl.pallas_call(
        functools.partial(_pipe_kernel, block=block, n_steps=n_steps),
        out_shape=jax.ShapeDtypeStruct((m_pad, n), jnp.float32),
        grid=(n_cores,),
        in_specs=[
            pl.BlockSpec(memory_space=pl.ANY),
            pl.BlockSpec((in_dim, n), lambda i: (0, 0)),
            pl.BlockSpec((1, n), lambda i: (0, 0)),
        ],
        out_specs=pl.BlockSpec(memory_space=pl.ANY),
        scratch_shapes=[
            pltpu.VMEM((in_dim, n), jnp.bfloat16),
            pltpu.VMEM((2, block, in_dim), jnp.float32),
            pltpu.VMEM((2, block, n), jnp.float32),
            pltpu.SemaphoreType.DMA((2,)),
            pltpu.SemaphoreType.DMA((2,)),
        ],
        compiler_params=pltpu.CompilerParams(
            dimension_semantics=("parallel",),
            vmem_limit_bytes=56 * 1024 * 1024,
        ),
    )(x_p, w, bias)

    return out[:batch] if m_pad != batch else out


def kernel(x, wt_padded, bias_padded):
    return _affine(x, wt_padded, bias_padded)
